# baseline (speedup 1.0000x reference)
import functools

import jax
import jax.numpy as jnp
from jax import lax
from jax.experimental import pallas as pl
from jax.experimental.pallas import tpu as pltpu
from jax.experimental.pallas import tpu_sc as plsc


def _kernel_sc(x, pos_table):
    B, S, D = x.shape
    info = plsc.get_sparse_core_info()
    NC, NS, L = info.num_cores, info.num_subcores, info.num_lanes
    NW = NC * NS
    RW = S // NW
    C = 16
    NCH = RW // C
    NI = B * NCH
    NB = 3

    mesh = plsc.VectorSubcoreMesh(core_axis_name="c", subcore_axis_name="s")

    @functools.partial(
        pl.kernel,
        mesh=mesh,
        out_type=jax.ShapeDtypeStruct((B, S, D), jnp.float32),
        compiler_params=pltpu.CompilerParams(
            use_tc_tiling_on_sc=True,
            disable_bounds_checks=True,
            disable_semaphore_checks=True,
        ),
        scratch_types=[pltpu.VMEM((RW, D), jnp.float32)]
        + [pltpu.VMEM((C, D), jnp.float32) for _ in range(NB)]
        + [pltpu.SemaphoreType.DMA for _ in range(2 * NB + RW // C)],
    )
    def run(x_hbm, pos_hbm, out_hbm, pos_v, *rest):
        bufs = rest[:NB]
        lsems = rest[NB : 2 * NB]
        ssems = rest[2 * NB : 3 * NB]
        psems = rest[3 * NB :]

        wid = lax.axis_index("s") * NC + lax.axis_index("c")
        r0 = wid * RW

        def item_cb(i):
            return i % NCH, i // NCH

        def load(i, k):
            c, b = item_cb(i)
            return pltpu.async_copy(
                x_hbm.at[b, pl.ds(r0 + c * C, C), :], bufs[k], lsems[k]
            )

        def load_pos(c):
            return pltpu.async_copy(
                pos_hbm.at[pl.ds(r0 + c * C, C), :],
                pos_v.at[pl.ds(c * C, C), :],
                psems[c],
            )

        pdescs = [load_pos(0)]
        ldesc = [None] * NB
        sdesc = [None] * NB
        for k in range(2):
            ldesc[k] = load(k, k)
        for cc in range(1, NCH):
            pdescs.append(load_pos(cc))

        for i in range(NI):
            k = i % NB
            c, b = item_cb(i)
            ldesc[k].wait()
            if i < NCH:
                pdescs[c].wait()

            buf = bufs[k]
            pbase = c * C

            @plsc.parallel_loop(0, C * D, L, unroll=8)
            def add_body(j):
                r = j // D
                col = j % D
                plsc.addupdate(
                    buf.at[r, pl.ds(col, L)], pos_v[pbase + r, pl.ds(col, L)]
                )

            sdesc[k] = pltpu.async_copy(
                buf, out_hbm.at[b, pl.ds(r0 + c * C, C), :], ssems[k]
            )

            ni = i + 2
            if ni < NI:
                nk = ni % NB
                if sdesc[nk] is not None:
                    sdesc[nk].wait()
                ldesc[nk] = load(ni, nk)

        for i in range(max(0, NI - NB), NI):
            sdesc[i % NB].wait()

    return run(x, pos_table)


def kernel(x, pos_table):
    return _kernel_sc(x, pos_table)


# speedup vs baseline: 1.0104x; 1.0104x over previous
import functools

import jax
import jax.numpy as jnp
from jax import lax
from jax.experimental import pallas as pl
from jax.experimental.pallas import tpu as pltpu
from jax.experimental.pallas import tpu_sc as plsc


def _kernel_sc(x, pos_table):
    B, S, D = x.shape
    info = plsc.get_sparse_core_info()
    NC, NS, L = info.num_cores, info.num_subcores, info.num_lanes
    NW = NC * NS
    RW = S // NW
    C = 16
    NCH = RW // C
    NI = B * NCH
    NB = 3

    mesh = plsc.VectorSubcoreMesh(core_axis_name="c", subcore_axis_name="s")

    @functools.partial(
        pl.kernel,
        mesh=mesh,
        out_type=jax.ShapeDtypeStruct((B, S, D), jnp.float32),
        compiler_params=pltpu.CompilerParams(
            use_tc_tiling_on_sc=True,
            disable_bounds_checks=True,
            disable_semaphore_checks=True,
        ),
        scratch_types=[pltpu.VMEM((RW, D), jnp.float32)]
        + [pltpu.VMEM((C, D), jnp.float32) for _ in range(NB)]
        + [pltpu.SemaphoreType.DMA for _ in range(2 * NB + RW // C)],
    )
    def run(x_hbm, pos_hbm, out_hbm, pos_v, *rest):
        bufs = rest[:NB]
        lsems = rest[NB : 2 * NB]
        ssems = rest[2 * NB : 3 * NB]
        psems = rest[3 * NB :]

        wid = lax.axis_index("s") * NC + lax.axis_index("c")
        r0 = wid * RW

        def item_cb(i):
            return i % NCH, i // NCH

        def load(i, k):
            c, b = item_cb(i)
            return pltpu.async_copy(
                x_hbm.at[b, pl.ds(r0 + c * C, C), :], bufs[k], lsems[k]
            )

        def load_pos(c):
            return pltpu.async_copy(
                pos_hbm.at[pl.ds(r0 + c * C, C), :],
                pos_v.at[pl.ds(c * C, C), :],
                psems[c],
            )

        pdescs = [load_pos(0)]
        ldesc = [None] * NB
        sdesc = [None] * NB
        for k in range(2):
            ldesc[k] = load(k, k)
        for cc in range(1, NCH):
            pdescs.append(load_pos(cc))

        def wait_load(i, k, c, b):
            pltpu.make_async_copy(
                x_hbm.at[b, pl.ds(r0 + c * C, C), :], bufs[k], lsems[k]
            ).wait()

        def wait_store(k, c, b):
            pltpu.make_async_copy(
                bufs[k], out_hbm.at[b, pl.ds(r0 + c * C, C), :], ssems[k]
            ).wait()

        def add_chunk(k, c):
            buf = bufs[k]
            pbase = c * C

            @plsc.parallel_loop(0, C * D, L, unroll=8)
            def add_body(j):
                r = j // D
                col = j % D
                plsc.addupdate(
                    buf.at[r, pl.ds(col, L)], pos_v[pbase + r, pl.ds(col, L)]
                )

        def store(i, k, c, b):
            return pltpu.async_copy(
                bufs[k], out_hbm.at[b, pl.ds(r0 + c * C, C), :], ssems[k]
            )

        for i in range(NCH):
            k = i % NB
            c, b = item_cb(i)
            ldesc[k].wait()
            pdescs[c].wait()
            add_chunk(k, c)
            sdesc[k] = store(i, k, c, b)
            ni = i + 2
            nk = ni % NB
            if sdesc[nk] is not None:
                sdesc[nk].wait()
            ldesc[nk] = load(ni, nk)

        NG = (NI - NCH - NB) // NB

        def group(g, carry):
            for t in range(NB):
                i = NCH + NB * g + t
                k = (NCH + t) % NB
                c = i % NCH
                b = i // NCH
                wait_load(i, k, c, b)
                add_chunk(k, c)
                store(i, k, c, b)
                ni = i + 2
                nk = (k + 2) % NB
                c2, b2 = ni % NCH, ni // NCH
                wait_store(nk, (i - 1) % NCH, (i - 1) // NCH)
                load(ni, nk)
            return carry

        lax.fori_loop(0, NG, group, 0)

        for i in range(NCH + NB * NG, NI):
            k = i % NB
            c, b = item_cb(i)
            wait_load(i, k, c, b)
            add_chunk(k, c)
            sdesc[k] = store(i, k, c, b)
            ni = i + 2
            if ni < NI:
                nk = ni % NB
                pj = ni - NB
                wait_store(nk, pj % NCH, pj // NCH)
                ldesc[nk] = load(ni, nk)

        for i in range(NI - NB, NI):
            sdesc[i % NB].wait()

    return run(x, pos_table)


def kernel(x, pos_table):
    return _kernel_sc(x, pos_table)
